# baseline (speedup 1.0000x reference)
import functools

import jax
import jax.numpy as jnp
from jax.experimental import pallas as pl

N = 10000
E = 160000
B = 4
INPUT_DIM = 2
NUM_UNITS = 64
MAX_DIFFUSION_STEP = 2
NUM_MATRICES = 2 * MAX_DIFFUSION_STEP + 1


def _proj(x2d, W, b, act):
    M, K = x2d.shape
    out = W.shape[1]
    blk = 2000
    b2 = b.reshape(1, out)

    def body(x_ref, w_ref, b_ref, o_ref):
        y = jnp.dot(x_ref[...], w_ref[...], preferred_element_type=jnp.float32)
        o_ref[...] = act(y + b_ref[...])

    return pl.pallas_call(
        body,
        grid=(M // blk,),
        in_specs=[
            pl.BlockSpec((blk, K), lambda i: (i, 0)),
            pl.BlockSpec((K, out), lambda i: (0, 0)),
            pl.BlockSpec((1, out), lambda i: (0, 0)),
        ],
        out_specs=pl.BlockSpec((blk, out), lambda i: (i, 0)),
        out_shape=jax.ShapeDtypeStruct((M, out), jnp.float32),
    )(x2d, W, b2)


def _gconv(x, W, b, src, dst, w1, w2, act):
    Bsz, Nn, C = x.shape
    x0 = jnp.transpose(x, (1, 2, 0)).reshape(Nn, C * Bsz)

    def smm0(v):
        return jnp.zeros_like(v).at[dst].add(w1[:, None] * v[src])

    def smm1(v):
        return jnp.zeros_like(v).at[src].add(w2[:, None] * v[dst])

    xs = [x0]
    for smm in (smm0, smm1):
        x1 = smm(x0)
        xs.append(x1)
        xk2, xk1 = x0, x1
        for _k in range(2, MAX_DIFFUSION_STEP + 1):
            x2 = 2.0 * smm(xk1) - xk2
            xs.append(x2)
            xk2, xk1 = xk1, x2
    xcat = jnp.stack(xs, axis=0).reshape(NUM_MATRICES, Nn, C, Bsz)
    xcat = jnp.transpose(xcat, (3, 1, 2, 0)).reshape(Bsz * Nn, C * NUM_MATRICES)
    return _proj(xcat, W, b, act).reshape(Bsz, Nn, -1)


def _dcgru_cell(x, h, Wg, bg, Wc, bc, src, dst, w1, w2):
    value = _gconv(jnp.concatenate([x, h], axis=-1), Wg, bg, src, dst, w1, w2,
                   jax.nn.sigmoid)
    r, u = jnp.split(value, 2, axis=-1)
    c = _gconv(jnp.concatenate([x, r * h], axis=-1), Wc, bc, src, dst, w1, w2,
               jnp.tanh)
    return u * h + (1.0 - u) * c


def kernel(inputs, hidden_state, src, dst, w1, w2, Wg0, bg0, Wc0, bc0, Wg1, bg1, Wc1, bc1):
    layer_params = [(Wg0, bg0, Wc0, bc0), (Wg1, bg1, Wc1, bc1)]
    output = inputs
    hs = []
    for l in range(2):
        Wg, bg, Wc, bc = layer_params[l]
        h = _dcgru_cell(output, hidden_state[l], Wg, bg, Wc, bc, src, dst, w1, w2)
        hs.append(h)
        output = h
    return (output, jnp.stack(hs))


# speedup vs baseline: 1.6995x; 1.6995x over previous
import functools

import jax
import jax.numpy as jnp
from jax import lax
from jax.experimental import pallas as pl
from jax.experimental.pallas import tpu as pltpu
from jax.experimental.pallas import tpu_sc as plsc

N = 10000
E = 160000
B = 4
INPUT_DIM = 2
NUM_UNITS = 64
MAX_DIFFUSION_STEP = 2
NUM_MATRICES = 2 * MAX_DIFFUSION_STEP + 1

_NC = 2
_NS = 16
_EPT = E // _NS
_EK = 200
_NB = _EPT // _EK
_NP = 10240
_RPT = _NP // _NS
_ZR = 64


@functools.cache
def _make_spmm(widths):
    nch = len(widths)
    uw = sorted(set(widths))
    mesh = plsc.VectorSubcoreMesh(core_axis_name="c", subcore_axis_name="s")

    scratch = []
    acc_ix, zbuf_ix, gbuf_ix = {}, {}, {}
    for w_ in uw:
        acc_ix[w_] = len(scratch)
        scratch.append(pltpu.VMEM_SHARED((_NP, w_), jnp.float32))
        zbuf_ix[w_] = len(scratch)
        scratch.append(pltpu.VMEM((_ZR, w_), jnp.float32))
        gbuf_ix[w_] = len(scratch)
        scratch.append(pltpu.VMEM((_EK, w_), jnp.float32))
    gi_ix = len(scratch); scratch.append(pltpu.VMEM((_EK,), jnp.int32))
    si_ix = len(scratch); scratch.append(pltpu.VMEM((_EK,), jnp.int32))
    wv_ix = len(scratch); scratch.append(pltpu.VMEM((_EK + 16,), jnp.float32))
    sem_ix = len(scratch); scratch.append(pltpu.SemaphoreType.DMA)

    @functools.partial(
        pl.kernel,
        out_type=tuple(jax.ShapeDtypeStruct((_NP, w_), jnp.float32) for w_ in widths),
        mesh=mesh,
        scratch_types=scratch,
    )
    def spmm(*refs):
        tbls = refs[:nch]
        gidx_h, sidx_h, w_h = refs[nch:nch + 3]
        outs = refs[nch + 3:nch + 3 + nch]
        scr = refs[nch + 3 + nch:]
        cid = lax.axis_index("c")
        sid = lax.axis_index("s")
        zv = jnp.zeros((16,), jnp.float32)

        for w_ in uw:
            zb = scr[zbuf_ix[w_]]

            def zrow(i, _, zb=zb, w_=w_):
                for c in range(w_ // 16):
                    zb[i, pl.ds(c * 16, 16)] = zv
                return 0

            lax.fori_loop(0, _ZR, zrow, 0)

        for j in range(nch):
            fc = widths[j]
            acc = scr[acc_ix[fc]]
            zb = scr[zbuf_ix[fc]]
            gb = scr[gbuf_ix[fc]]
            gi, si, wv = scr[gi_ix], scr[si_ix], scr[wv_ix]
            sem = scr[sem_ix]
            tbl, out = tbls[j], outs[j]

            @pl.when(cid == (j % _NC))
            def _(acc=acc, zb=zb, gb=gb, tbl=tbl, out=out, fc=fc):
                row0 = sid * _RPT
                for z in range(_RPT // _ZR):
                    pltpu.sync_copy(zb, acc.at[pl.ds(row0 + z * _ZR, _ZR)])
                plsc.subcore_barrier()

                ebase = sid * _EPT

                def blk(b, _):
                    base = ebase + b * _EK
                    pltpu.sync_copy(gidx_h.at[pl.ds(base, _EK)], gi)
                    pltpu.sync_copy(sidx_h.at[pl.ds(base, _EK)], si)
                    pltpu.sync_copy(w_h.at[pl.ds(base, _EK)], wv.at[pl.ds(0, _EK)])
                    pltpu.async_copy(tbl.at[gi], gb, sem).wait()

                    def erow(i, _):
                        ws = wv[pl.ds(i, 16)][0]
                        for c in range(fc // 16):
                            sl = pl.ds(c * 16, 16)
                            gb[i, sl] = gb[i, sl] * ws
                        return 0

                    lax.fori_loop(0, _EK, erow, 0)
                    pltpu.sync_copy(gb, acc.at[si], add=True)
                    return 0

                lax.fori_loop(0, _NB, blk, 0)
                plsc.subcore_barrier()
                pltpu.sync_copy(acc.at[pl.ds(row0, _RPT)],
                                out.at[pl.ds(row0, _RPT)])
                plsc.subcore_barrier()

    return spmm


def _smm_sc(v, gidx, sidx, w):
    wtot = v.shape[1]
    widths = (128,) * (wtot // 128)
    fn = _make_spmm(widths)
    tbls, c = [], 0
    for fc in widths:
        tbls.append(v[:, c:c + fc])
        c += fc
    outs = fn(*tbls, gidx, sidx, w)
    if not isinstance(outs, (tuple, list)):
        outs = (outs,)
    outs = [o[:N] for o in outs]
    if len(widths) == 1:
        return outs[0]
    return jnp.concatenate(outs, axis=1)


def _proj(x2d, W, b, act):
    M, K = x2d.shape
    out = W.shape[1]
    blk = 2000
    b2 = b.reshape(1, out)

    def body(x_ref, w_ref, b_ref, o_ref):
        y = jnp.dot(x_ref[...], w_ref[...], preferred_element_type=jnp.float32)
        o_ref[...] = act(y + b_ref[...])

    return pl.pallas_call(
        body,
        grid=(M // blk,),
        in_specs=[
            pl.BlockSpec((blk, K), lambda i: (i, 0)),
            pl.BlockSpec((K, out), lambda i: (0, 0)),
            pl.BlockSpec((1, out), lambda i: (0, 0)),
        ],
        out_specs=pl.BlockSpec((blk, out), lambda i: (i, 0)),
        out_shape=jax.ShapeDtypeStruct((M, out), jnp.float32),
    )(x2d, W, b2)


def _gconv(x, W, b, src, dst, w1, w2, act):
    Bsz, Nn, C = x.shape
    x0 = jnp.transpose(x, (1, 2, 0)).reshape(Nn, C * Bsz)
    wtot = C * Bsz
    wpad = (-wtot) % 128
    if wpad:
        x0 = jnp.pad(x0, ((0, 0), (0, wpad)))

    def smm0(v):
        return _smm_sc(v, src, dst, w1)

    def smm1(v):
        return _smm_sc(v, dst, src, w2)

    xs = [x0]
    for smm in (smm0, smm1):
        x1 = smm(x0)
        xs.append(x1)
        xk2, xk1 = x0, x1
        for _k in range(2, MAX_DIFFUSION_STEP + 1):
            x2 = 2.0 * smm(xk1) - xk2
            xs.append(x2)
            xk2, xk1 = xk1, x2
    if wpad:
        xs = [a[:, :wtot] for a in xs]
    xcat = jnp.stack(xs, axis=0).reshape(NUM_MATRICES, Nn, C, Bsz)
    xcat = jnp.transpose(xcat, (3, 1, 2, 0)).reshape(Bsz * Nn, C * NUM_MATRICES)
    return _proj(xcat, W, b, act).reshape(Bsz, Nn, -1)


def _dcgru_cell(x, h, Wg, bg, Wc, bc, src, dst, w1, w2):
    value = _gconv(jnp.concatenate([x, h], axis=-1), Wg, bg, src, dst, w1, w2,
                   jax.nn.sigmoid)
    r, u = jnp.split(value, 2, axis=-1)
    c = _gconv(jnp.concatenate([x, r * h], axis=-1), Wc, bc, src, dst, w1, w2,
               jnp.tanh)
    return u * h + (1.0 - u) * c


def kernel(inputs, hidden_state, src, dst, w1, w2, Wg0, bg0, Wc0, bc0, Wg1, bg1, Wc1, bc1):
    layer_params = [(Wg0, bg0, Wc0, bc0), (Wg1, bg1, Wc1, bc1)]
    output = inputs
    hs = []
    for l in range(2):
        Wg, bg, Wc, bc = layer_params[l]
        h = _dcgru_cell(output, hidden_state[l], Wg, bg, Wc, bc, src, dst, w1, w2)
        hs.append(h)
        output = h
    return (output, jnp.stack(hs))
